# Initial kernel scaffold; baseline (speedup 1.0000x reference)
import jax
import jax.numpy as jnp
from jax.experimental import pallas as pl


def kernel(piece_positions, king_positions, input_weights, input_bias, w1, b1, w2, b2, w_out, b_out):
    raise NotImplementedError("write your pallas kernel here")


# speedup vs baseline: 19.9273x; 19.9273x over previous
import jax
import jax.numpy as jnp
from jax.experimental import pallas as pl
from jax.experimental.pallas import tpu as pltpu

B = 1024
F = 640
D = 256
NK = 64
PP_TILE = 64


def _ppsum_body(pp_ref, out_ref):
    out_ref[...] = jnp.sum(pp_ref[...], axis=1).astype(jnp.float32)


def _main_body(ppsum_ref, w_ref, kings_ref, bias_ref, w1_ref, b1_ref,
               w2_ref, b2_ref, wout_ref, bout_ref, out_ref, xacc_ref):
    k = pl.program_id(0)
    kings = kings_ref[...]
    m = (kings == k).astype(jnp.float32)
    msum = m[:, 0:1] + m[:, 1:2]
    wk = w_ref[0]
    masked = ppsum_ref[...] * msum
    contrib = jnp.dot(masked, wk[:F, :],
                      preferred_element_type=jnp.float32)
    contrib = contrib + msum * wk[F:F + 1, :]

    @pl.when(k == 0)
    def _init():
        xacc_ref[...] = contrib

    @pl.when(k > 0)
    def _acc():
        xacc_ref[...] = xacc_ref[...] + contrib

    @pl.when(k == NK - 1)
    def _tail():
        x = xacc_ref[...] + bias_ref[...]
        x = jnp.clip(x, 0.0, 127.0)
        w1s = w1_ref[...][:, :D] + w1_ref[...][:, D:]
        h = jax.lax.dot_general(x, w1s, (((1,), (1,)), ((), ())),
                                preferred_element_type=jnp.float32)
        h = h + b1_ref[...]
        h = jnp.clip(jnp.floor(h * (1.0 / 64.0)), 0.0, 127.0)
        h = jax.lax.dot_general(h, w2_ref[...], (((1,), (1,)), ((), ())),
                                preferred_element_type=jnp.float32)
        h = h + b2_ref[...]
        h = jnp.clip(jnp.floor(h * (1.0 / 64.0)), 0.0, 127.0)
        v = jnp.sum(h * wout_ref[...]) + bout_ref[...]
        out_ref[...] = jnp.floor(v * (1.0 / 16.0))


def kernel(piece_positions, king_positions, input_weights, input_bias,
           w1, b1, w2, b2, w_out, b_out):
    ppsum = pl.pallas_call(
        _ppsum_body,
        grid=(B // PP_TILE,),
        in_specs=[pl.BlockSpec((PP_TILE, 64, F), lambda i: (i, 0, 0))],
        out_specs=pl.BlockSpec((PP_TILE, F), lambda i: (i, 0)),
        out_shape=jax.ShapeDtypeStruct((B, F), jnp.float32),
    )(piece_positions)

    out = pl.pallas_call(
        _main_body,
        grid=(NK,),
        in_specs=[
            pl.BlockSpec((B, F), lambda k: (0, 0)),
            pl.BlockSpec((1, F + 1, D), lambda k: (k, 0, 0)),
            pl.BlockSpec((B, 2), lambda k: (0, 0)),
            pl.BlockSpec((1, D), lambda k: (0, 0)),
            pl.BlockSpec((32, 2 * D), lambda k: (0, 0)),
            pl.BlockSpec((1, 32), lambda k: (0, 0)),
            pl.BlockSpec((32, 32), lambda k: (0, 0)),
            pl.BlockSpec((1, 32), lambda k: (0, 0)),
            pl.BlockSpec((1, 32), lambda k: (0, 0)),
            pl.BlockSpec((1, 1), lambda k: (0, 0)),
        ],
        out_specs=pl.BlockSpec((1, 1), lambda k: (0, 0)),
        out_shape=jax.ShapeDtypeStruct((1, 1), jnp.float32),
        scratch_shapes=[pltpu.VMEM((B, D), jnp.float32)],
    )(
        ppsum,
        input_weights,
        king_positions,
        input_bias.reshape(1, D),
        w1,
        b1.reshape(1, 32),
        w2,
        b2.reshape(1, 32),
        w_out.reshape(1, 32),
        b_out.reshape(1, 1),
    )
    return out.reshape((1,))


# speedup vs baseline: 20.2839x; 1.0179x over previous
import jax
import jax.numpy as jnp
from jax.experimental import pallas as pl
from jax.experimental.pallas import tpu as pltpu

B = 1024
F = 640
D = 256
NK = 64
KB = 4
PP_TILE = 128


def _ppsum_body(pp_ref, out_ref):
    out_ref[...] = jnp.sum(pp_ref[...], axis=1).astype(jnp.bfloat16)


def _main_body(ppsum_ref, w_ref, kings_ref, bias_ref, w1_ref, b1_ref,
               w2_ref, b2_ref, wout_ref, bout_ref, out_ref, xacc_ref):
    step = pl.program_id(0)
    kings = kings_ref[...]
    pp = ppsum_ref[...]

    acc = jnp.zeros((B, D), jnp.float32)
    for j in range(KB):
        k = step * KB + j
        wk = w_ref[j]
        m = (kings == k).astype(jnp.float32)
        msum = m[:, 0:1] + m[:, 1:2]
        wmat = wk[:F, :]
        whi = jnp.floor(wmat * (1.0 / 256.0))
        wlo = wmat - whi * 256.0
        zhi = jax.lax.dot_general(pp, whi.astype(jnp.bfloat16),
                                  (((1,), (0,)), ((), ())),
                                  preferred_element_type=jnp.float32)
        zlo = jax.lax.dot_general(pp, wlo.astype(jnp.bfloat16),
                                  (((1,), (0,)), ((), ())),
                                  preferred_element_type=jnp.float32)
        acc = acc + msum * (zhi * 256.0 + zlo + wk[F:F + 1, :])

    @pl.when(step == 0)
    def _init():
        xacc_ref[...] = acc

    @pl.when(step > 0)
    def _acc():
        xacc_ref[...] = xacc_ref[...] + acc

    @pl.when(step == NK // KB - 1)
    def _tail():
        x = xacc_ref[...] + bias_ref[...]
        x = jnp.clip(x, 0.0, 127.0)
        w1s = w1_ref[...][:, :D] + w1_ref[...][:, D:]
        h = jax.lax.dot_general(x, w1s, (((1,), (1,)), ((), ())),
                                preferred_element_type=jnp.float32)
        h = h + b1_ref[...]
        h = jnp.clip(jnp.floor(h * (1.0 / 64.0)), 0.0, 127.0)
        h = jax.lax.dot_general(h, w2_ref[...], (((1,), (1,)), ((), ())),
                                preferred_element_type=jnp.float32)
        h = h + b2_ref[...]
        h = jnp.clip(jnp.floor(h * (1.0 / 64.0)), 0.0, 127.0)
        v = jnp.sum(h * wout_ref[...]) + bout_ref[...]
        out_ref[...] = jnp.floor(v * (1.0 / 16.0))


def kernel(piece_positions, king_positions, input_weights, input_bias,
           w1, b1, w2, b2, w_out, b_out):
    ppsum = pl.pallas_call(
        _ppsum_body,
        grid=(B // PP_TILE,),
        in_specs=[pl.BlockSpec((PP_TILE, 64, F), lambda i: (i, 0, 0))],
        out_specs=pl.BlockSpec((PP_TILE, F), lambda i: (i, 0)),
        out_shape=jax.ShapeDtypeStruct((B, F), jnp.bfloat16),
    )(piece_positions)

    out = pl.pallas_call(
        _main_body,
        grid=(NK // KB,),
        in_specs=[
            pl.BlockSpec((B, F), lambda s: (0, 0)),
            pl.BlockSpec((KB, F + 1, D), lambda s: (s, 0, 0)),
            pl.BlockSpec((B, 2), lambda s: (0, 0)),
            pl.BlockSpec((1, D), lambda s: (0, 0)),
            pl.BlockSpec((32, 2 * D), lambda s: (0, 0)),
            pl.BlockSpec((1, 32), lambda s: (0, 0)),
            pl.BlockSpec((32, 32), lambda s: (0, 0)),
            pl.BlockSpec((1, 32), lambda s: (0, 0)),
            pl.BlockSpec((1, 32), lambda s: (0, 0)),
            pl.BlockSpec((1, 1), lambda s: (0, 0)),
        ],
        out_specs=pl.BlockSpec((1, 1), lambda s: (0, 0)),
        out_shape=jax.ShapeDtypeStruct((1, 1), jnp.float32),
        scratch_shapes=[pltpu.VMEM((B, D), jnp.float32)],
    )(
        ppsum,
        input_weights,
        king_positions,
        input_bias.reshape(1, D),
        w1,
        b1.reshape(1, 32),
        w2,
        b2.reshape(1, 32),
        w_out.reshape(1, 32),
        b_out.reshape(1, 1),
    )
    return out.reshape((1,))


# speedup vs baseline: 24.3818x; 1.2020x over previous
import jax
import jax.numpy as jnp
from jax.experimental import pallas as pl
from jax.experimental.pallas import tpu as pltpu

B = 1024
F = 640
D = 256
NK = 64
KB = 8
PP_TILE = 128


def _ppsum_body(pp_ref, out_ref):
    out_ref[...] = jnp.sum(pp_ref[...], axis=1).astype(jnp.float32)


def _main_body(ppsum_ref, w_ref, kings_ref, bias_ref, w1_ref, b1_ref,
               w2_ref, b2_ref, wout_ref, bout_ref, out_ref, xacc_ref):
    step = pl.program_id(0)
    kings = kings_ref[...]
    pp = ppsum_ref[...]

    acc = jnp.zeros((B, D), jnp.float32)
    for j in range(KB):
        k = step * KB + j
        wk = w_ref[j]
        m = (kings == k).astype(jnp.float32)
        msum = m[:, 0:1] + m[:, 1:2]
        z = jax.lax.dot_general(pp, wk[:F, :],
                                (((1,), (0,)), ((), ())),
                                preferred_element_type=jnp.float32)
        acc = acc + msum * (z + wk[F:F + 1, :])

    @pl.when(step == 0)
    def _init():
        xacc_ref[...] = acc

    @pl.when(step > 0)
    def _acc():
        xacc_ref[...] = xacc_ref[...] + acc

    @pl.when(step == NK // KB - 1)
    def _tail():
        x = xacc_ref[...] + bias_ref[...]
        x = jnp.clip(x, 0.0, 127.0)
        w1s = w1_ref[...][:, :D] + w1_ref[...][:, D:]
        h = jax.lax.dot_general(x, w1s, (((1,), (1,)), ((), ())),
                                preferred_element_type=jnp.float32)
        h = h + b1_ref[...]
        h = jnp.clip(jnp.floor(h * (1.0 / 64.0)), 0.0, 127.0)
        h = jax.lax.dot_general(h, w2_ref[...], (((1,), (1,)), ((), ())),
                                preferred_element_type=jnp.float32)
        h = h + b2_ref[...]
        h = jnp.clip(jnp.floor(h * (1.0 / 64.0)), 0.0, 127.0)
        v = jnp.sum(h * wout_ref[...]) + bout_ref[...]
        out_ref[...] = jnp.floor(v * (1.0 / 16.0))


def kernel(piece_positions, king_positions, input_weights, input_bias,
           w1, b1, w2, b2, w_out, b_out):
    ppsum = pl.pallas_call(
        _ppsum_body,
        grid=(B // PP_TILE,),
        in_specs=[pl.BlockSpec((PP_TILE, 64, F), lambda i: (i, 0, 0))],
        out_specs=pl.BlockSpec((PP_TILE, F), lambda i: (i, 0)),
        out_shape=jax.ShapeDtypeStruct((B, F), jnp.float32),
    )(piece_positions)

    out = pl.pallas_call(
        _main_body,
        grid=(NK // KB,),
        in_specs=[
            pl.BlockSpec((B, F), lambda s: (0, 0)),
            pl.BlockSpec((KB, F + 1, D), lambda s: (s, 0, 0)),
            pl.BlockSpec((B, 2), lambda s: (0, 0)),
            pl.BlockSpec((1, D), lambda s: (0, 0)),
            pl.BlockSpec((32, 2 * D), lambda s: (0, 0)),
            pl.BlockSpec((1, 32), lambda s: (0, 0)),
            pl.BlockSpec((32, 32), lambda s: (0, 0)),
            pl.BlockSpec((1, 32), lambda s: (0, 0)),
            pl.BlockSpec((1, 32), lambda s: (0, 0)),
            pl.BlockSpec((1, 1), lambda s: (0, 0)),
        ],
        out_specs=pl.BlockSpec((1, 1), lambda s: (0, 0)),
        out_shape=jax.ShapeDtypeStruct((1, 1), jnp.float32),
        scratch_shapes=[pltpu.VMEM((B, D), jnp.float32)],
    )(
        ppsum,
        input_weights,
        king_positions,
        input_bias.reshape(1, D),
        w1,
        b1.reshape(1, 32),
        w2,
        b2.reshape(1, 32),
        w_out.reshape(1, 32),
        b_out.reshape(1, 1),
    )
    return out.reshape((1,))
